# baseline (speedup 1.0000x reference)
import jax
import jax.numpy as jnp
from jax import lax
from jax.experimental import pallas as pl
from jax.experimental.pallas import tpu as pltpu
from jax.experimental.pallas import tpu_sc as plsc

VOCAB = 100000
D_MODEL = 768
MAX_SEQ = 512
BATCH = 64
SEQ = 512

NC = 2
NS = 16
LANES = 16
NW = NC * NS
ROWS = BATCH * SEQ
ROWS_PER_W = ROWS // NW
CHUNK = 64
NCHUNK = ROWS_PER_W // CHUNK
DV = D_MODEL // LANES


def _body(ids_hbm, tok_hbm, pos_hbm, out_hbm, idx_v, rows_v, pos_v, sem):
    wid = lax.axis_index("s") * NC + lax.axis_index("c")
    base = wid * ROWS_PER_W
    pltpu.sync_copy(ids_hbm.at[pl.ds(base, ROWS_PER_W)], idx_v)

    def chunk_step(j, carry):
        r0 = j * CHUNK
        gather = pltpu.async_copy(
            tok_hbm.at[idx_v.at[pl.ds(r0, CHUNK)]], rows_v, sem)
        s0 = lax.rem(r0, SEQ)
        pltpu.sync_copy(pos_hbm.at[pl.ds(s0, CHUNK)], pos_v)
        gather.wait()

        def row_step(r, c):
            for k in range(DV):
                sl = pl.ds(k * LANES, LANES)
                rows_v[r, sl] = rows_v[r, sl] + pos_v[r, sl]
            return c

        lax.fori_loop(0, CHUNK, row_step, 0)
        pltpu.sync_copy(rows_v, out_hbm.at[pl.ds(base + r0, CHUNK)])
        return carry

    lax.fori_loop(0, NCHUNK, chunk_step, 0)


@jax.jit
def _run(ids_flat, token_table, position_table):
    mesh = plsc.VectorSubcoreMesh(core_axis_name="c", subcore_axis_name="s")
    return pl.kernel(
        _body,
        out_type=jax.ShapeDtypeStruct((ROWS, D_MODEL), jnp.float32),
        mesh=mesh,
        scratch_types=[
            pltpu.VMEM((ROWS_PER_W,), jnp.int32),
            pltpu.VMEM((CHUNK, D_MODEL), jnp.float32),
            pltpu.VMEM((CHUNK, D_MODEL), jnp.float32),
            pltpu.SemaphoreType.DMA,
        ],
    )(ids_flat, token_table, position_table)


def kernel(input_ids, token_table, position_table):
    ids_flat = input_ids.reshape(-1).astype(jnp.int32)
    out = _run(ids_flat, token_table, position_table)
    return out.reshape(BATCH, SEQ, D_MODEL)


# speedup vs baseline: 1.9512x; 1.9512x over previous
import jax
import jax.numpy as jnp
from jax import lax
from jax.experimental import pallas as pl
from jax.experimental.pallas import tpu as pltpu
from jax.experimental.pallas import tpu_sc as plsc

VOCAB = 100000
D_MODEL = 768
MAX_SEQ = 512
BATCH = 64
SEQ = 512

NC = 2
NS = 16
LANES = 16
NW = NC * NS
ROWS = BATCH * SEQ
ROWS_PER_W = ROWS // NW
S_PER_W = SEQ // NW
CHUNK = BATCH
NCHUNK = ROWS_PER_W // CHUNK
DV = D_MODEL // LANES


def _body(ids_hbm, oidx_hbm, tok_hbm, pos_hbm, out_hbm,
          idx_v, oidx_v, pos_v, rows0, rows1, g0, g1, s0, s1):
    wid = lax.axis_index("s") * NC + lax.axis_index("c")
    base = wid * ROWS_PER_W
    pltpu.sync_copy(ids_hbm.at[pl.ds(base, ROWS_PER_W)], idx_v)
    pltpu.sync_copy(oidx_hbm.at[wid], oidx_v)
    pltpu.sync_copy(pos_hbm.at[pl.ds(wid * S_PER_W, S_PER_W)], pos_v)

    bufs = (rows0, rows1)
    gsems = (g0, g1)
    ssems = (s0, s1)

    def start_gather(j, buf, sem):
        pltpu.async_copy(tok_hbm.at[idx_v.at[pl.ds(j * CHUNK, CHUNK)]],
                         buf, sem)

    start_gather(0, rows0, g0)

    def pair_step(p, carry):
        for b in range(2):
            j = p * 2 + b
            other = 1 - b

            @pl.when(j + 1 < NCHUNK)
            def _prefetch():
                @pl.when(j >= 1)
                def _drain_store():
                    pltpu.make_async_copy(
                        bufs[other], out_hbm.at[oidx_v.at[j - 1]],
                        ssems[other]).wait()
                start_gather(j + 1, bufs[other], gsems[other])

            pltpu.make_async_copy(
                tok_hbm.at[idx_v.at[pl.ds(j * CHUNK, CHUNK)]],
                bufs[b], gsems[b]).wait()

            pvecs = [pos_v[j, pl.ds(k * LANES, LANES)] for k in range(DV)]

            def row_step(r, c):
                for k in range(DV):
                    sl = pl.ds(k * LANES, LANES)
                    bufs[b][r, sl] = bufs[b][r, sl] + pvecs[k]
                return c

            lax.fori_loop(0, CHUNK, row_step, 0)
            pltpu.async_copy(bufs[b], out_hbm.at[oidx_v.at[j]], ssems[b])
        return carry

    lax.fori_loop(0, NCHUNK // 2, pair_step, 0)
    pltpu.make_async_copy(rows0, out_hbm.at[oidx_v.at[NCHUNK - 2]], s0).wait()
    pltpu.make_async_copy(rows1, out_hbm.at[oidx_v.at[NCHUNK - 1]], s1).wait()


@jax.jit
def _run(ids_t, out_idx, token_table, position_table):
    mesh = plsc.VectorSubcoreMesh(core_axis_name="c", subcore_axis_name="s",
                                  num_cores=NC, num_subcores=NS)
    return pl.kernel(
        _body,
        out_type=jax.ShapeDtypeStruct((ROWS, D_MODEL), jnp.float32),
        mesh=mesh,
        scratch_types=[
            pltpu.VMEM((ROWS_PER_W,), jnp.int32),
            pltpu.VMEM((NCHUNK, CHUNK), jnp.int32),
            pltpu.VMEM((S_PER_W, D_MODEL), jnp.float32),
            pltpu.VMEM((CHUNK, D_MODEL), jnp.float32),
            pltpu.VMEM((CHUNK, D_MODEL), jnp.float32),
            pltpu.SemaphoreType.DMA,
            pltpu.SemaphoreType.DMA,
            pltpu.SemaphoreType.DMA,
            pltpu.SemaphoreType.DMA,
        ],
    )(ids_t, out_idx, token_table, position_table)


def kernel(input_ids, token_table, position_table):
    ids_t = input_ids.T.reshape(-1).astype(jnp.int32)
    s_ix = jnp.arange(SEQ, dtype=jnp.int32)
    b_ix = jnp.arange(BATCH, dtype=jnp.int32)
    out_idx = (b_ix[None, :] * SEQ + s_ix[:, None]).reshape(NW, NCHUNK, CHUNK)
    out = _run(ids_t, out_idx, token_table, position_table)
    return out.reshape(BATCH, SEQ, D_MODEL)
